# baseline (speedup 1.0000x reference)
import functools

import jax
import jax.numpy as jnp
import numpy as np
from jax import lax
from jax.experimental import pallas as pl
from jax.experimental.pallas import tpu as pltpu
from jax.experimental.pallas import tpu_sc as plsc

N_NODES = 50000
N_EDGES = 800000
DIM = 64
N_CENTERS = 5
CUTOFF = 5.0
N_CONV = 3

NC = 2
NS = 16
HALF = 32

KT = 256
KINT = KT - 8
SCALE = KINT / CUTOFF

SUB = 100
SPC = 4
CHUNK = SUB * SPC
EPT = N_EDGES // NS
STEPS = EPT // CHUNK
ROWS_PT = EPT // SUB
RPT = 3128
RPT_LAST = N_NODES - (NS - 1) * RPT
ZFULL = RPT // CHUNK

_BCAST_DNUMS = lax.GatherDimensionNumbers(
    offset_dims=(), collapsed_slice_dims=(0,), start_index_map=(0,))


def _lane_bcast(v, i):
    idx = jnp.full((16, 1), i, jnp.int32)
    return lax.gather(v, idx, _BCAST_DNUMS, (1,),
                      mode=lax.GatherScatterMode.PROMISE_IN_BOUNDS)

_LOG2 = float(np.log(2.0))


def _bf16_bits(x):
    b = lax.bitcast_convert_type(x, jnp.int32)
    return lax.shift_right_logical(
        b + 0x7FFF + (lax.shift_right_logical(b, 16) & 1), 16)


def _pack_bf16(x, y):
    return _bf16_bits(x) | lax.shift_left(_bf16_bits(y), 16)


def _softplus(x):
    return jnp.maximum(x, 0.0) + jnp.log1p(jnp.exp(-jnp.abs(x)))


def _softplus_b(x):
    return 2.0 * _softplus(0.5 * x)


_BN = 5000

_TDOT = (((0,), (0,)), ((), ()))


def _table_body(w1_ref, b1_ref, w2_ref, b2_ref, out_ref):
    gap = CUTOFF / (N_CENTERS - 1)
    centers = lax.broadcasted_iota(
        jnp.int32, (N_CENTERS, 1), 0).astype(jnp.float32) * gap
    dk = lax.broadcasted_iota(
        jnp.int32, (1, KT), 1).astype(jnp.float32) * (1.0 / SCALE)
    rbfT = jnp.exp((-1.0 / gap) * (dk - centers) ** 2)
    for l in range(N_CONV):
        hpT = lax.dot_general(w1_ref[l], rbfT, _TDOT,
                              preferred_element_type=jnp.float32)
        hpT = _softplus_b(hpT + b1_ref[l])
        hh = lax.dot_general(hpT, w2_ref[l], _TDOT,
                             preferred_element_type=jnp.float32)
        hh = hh + b2_ref[l][None, :]
        dd = jnp.concatenate(
            [hh[1:] - hh[:-1], jnp.zeros((1, DIM), jnp.float32)], axis=0)
        for c in range(NC):
            t = hh[:, c * HALF:(c + 1) * HALF]
            d = dd[:, c * HALF:(c + 1) * HALF]
            out_ref[l, c] = jnp.concatenate(
                [_pack_bf16(t[:, :16], t[:, 16:]),
                 _pack_bf16(d[:, :16], d[:, 16:])], axis=1)


def _embed_body(nt_ref, emb_ref, out_ref):
    tn = emb_ref.shape[0]
    ids = lax.broadcasted_iota(jnp.int32, (tn, 1), 0)
    nt = nt_ref[0]
    ohT = (nt == ids).astype(jnp.float32)
    out_ref[...] = lax.dot_general(ohT, emb_ref[...], _TDOT,
                                   preferred_element_type=jnp.float32)


def _project_body(node_ref, w1_ref, out_ref):
    nn = jnp.dot(node_ref[...], w1_ref[...], preferred_element_type=jnp.float32)
    for c in range(NC):
        half = nn[:, c * HALF:(c + 1) * HALF]
        out_ref[c] = _pack_bf16(half[:, :16], half[:, 16:])


def _update_body(node_ref, agg_ref, w2_ref, b2_ref, w3_ref, b3_ref, out_ref):
    agg = jnp.concatenate([agg_ref[0], agg_ref[1]], axis=1)
    cf = _softplus_b(
        jnp.dot(agg, w2_ref[...], preferred_element_type=jnp.float32)
        + b2_ref[...][None, :])
    out_ref[...] = (node_ref[...]
                    + jnp.dot(cf, w3_ref[...], preferred_element_type=jnp.float32)
                    + b3_ref[...][None, :])


def _readout_body(node_ref, d1w_ref, d1b_ref, d2w_ref, d2b_ref, out_ref):
    atom = _softplus(
        jnp.dot(node_ref[...], d1w_ref[...], preferred_element_type=jnp.float32)
        + d1b_ref[...][None, :]) - _LOG2
    res = jnp.dot(atom, d2w_ref[...], preferred_element_type=jnp.float32)
    part = jnp.sum(res) + node_ref.shape[0] * d2b_ref[0]

    @pl.when(pl.program_id(0) == 0)
    def _():
        out_ref[...] = jnp.zeros_like(out_ref)

    out_ref[...] += jnp.reshape(part, (1, 1))


def _full(shape):
    return pl.BlockSpec(shape, lambda i: tuple(0 for _ in shape))


@functools.lru_cache(maxsize=None)
def _tc_calls(type_num):
    table = pl.pallas_call(
        _table_body,
        grid=(1,),
        in_specs=[
            _full((N_CONV, N_CENTERS, DIM)),
            _full((N_CONV, DIM, 1)),
            _full((N_CONV, DIM, DIM)),
            _full((N_CONV, DIM)),
        ],
        out_specs=pl.BlockSpec((N_CONV, NC, KT, HALF), lambda i: (0, 0, 0, 0)),
        out_shape=jax.ShapeDtypeStruct((N_CONV, NC, KT, HALF), jnp.int32),
    )
    embed = pl.pallas_call(
        _embed_body,
        grid=(N_NODES // _BN,),
        in_specs=[
            pl.BlockSpec((1, 1, _BN), lambda i: (i, 0, 0)),
            _full((type_num, DIM)),
        ],
        out_specs=pl.BlockSpec((_BN, DIM), lambda i: (i, 0)),
        out_shape=jax.ShapeDtypeStruct((N_NODES, DIM), jnp.float32),
    )
    project = pl.pallas_call(
        _project_body,
        grid=(N_NODES // _BN,),
        in_specs=[
            pl.BlockSpec((_BN, DIM), lambda i: (i, 0)),
            _full((DIM, DIM)),
        ],
        out_specs=pl.BlockSpec((NC, _BN, 16), lambda i: (0, i, 0)),
        out_shape=jax.ShapeDtypeStruct((NC, N_NODES, 16), jnp.int32),
    )
    update = pl.pallas_call(
        _update_body,
        grid=(N_NODES // _BN,),
        in_specs=[
            pl.BlockSpec((_BN, DIM), lambda i: (i, 0)),
            pl.BlockSpec((NC, _BN, HALF), lambda i: (0, i, 0)),
            _full((DIM, DIM)),
            _full((DIM,)),
            _full((DIM, DIM)),
            _full((DIM,)),
        ],
        out_specs=pl.BlockSpec((_BN, DIM), lambda i: (i, 0)),
        out_shape=jax.ShapeDtypeStruct((N_NODES, DIM), jnp.float32),
    )
    readout = pl.pallas_call(
        _readout_body,
        grid=(N_NODES // _BN,),
        in_specs=[
            pl.BlockSpec((_BN, DIM), lambda i: (i, 0)),
            _full((DIM, DIM)),
            _full((DIM,)),
            _full((DIM, 1)),
            _full((1,)),
        ],
        out_specs=pl.BlockSpec((1, 1), lambda i: (0, 0)),
        out_shape=jax.ShapeDtypeStruct((1, 1), jnp.float32),
    )
    return table, embed, project, update, readout


def _edge_body(layer, nn_hbm, tbl_hbm, d_hbm, src_hbm, dst_hbm, agg_hbm,
               acc, tbl_v, idxs, idxd, nbuf, nbin, dbuf, kbuf, fbuf, sem):
    c = lax.axis_index("c")
    s = lax.axis_index("s")
    z16 = jnp.zeros((16,), jnp.float32)

    pltpu.sync_copy(tbl_hbm.at[layer].at[c], tbl_v)

    def zbody(i, carry):
        nbuf[i, pl.ds(0, 16)] = z16
        nbuf[i, pl.ds(16, 16)] = z16
        return carry

    lax.fori_loop(0, CHUNK, zbody, 0)
    r0 = s * RPT

    def zcopy(k, carry):
        pltpu.sync_copy(nbuf, acc.at[pl.ds(r0 + k * CHUNK, CHUNK)])
        return carry

    lax.fori_loop(0, ZFULL, zcopy, 0)

    @pl.when(s < NS - 1)
    def _():
        pltpu.sync_copy(nbuf.at[pl.ds(0, RPT - ZFULL * CHUNK)],
                        acc.at[pl.ds(r0 + ZFULL * CHUNK, RPT - ZFULL * CHUNK)])

    @pl.when(s == NS - 1)
    def _():
        pltpu.sync_copy(
            nbuf.at[pl.ds(0, RPT_LAST - ZFULL * CHUNK)],
            acc.at[pl.ds(r0 + ZFULL * CHUNK, RPT_LAST - ZFULL * CHUNK)])

    plsc.subcore_barrier()

    col0 = lax.broadcasted_iota(jnp.int32, (16,), 0)
    col1 = col0 + 16

    def ebody(k, carry):
        row0 = s * ROWS_PT + k * SPC
        e0 = s * EPT + k * CHUNK
        pltpu.sync_copy(src_hbm.at[pl.ds(row0, SPC)], idxs)
        pltpu.sync_copy(dst_hbm.at[pl.ds(row0, SPC)], idxd)
        pltpu.sync_copy(d_hbm.at[pl.ds(e0, CHUNK)], dbuf)
        cps = [
            pltpu.async_copy(nn_hbm.at[c].at[idxs.at[j]],
                             nbin.at[pl.ds(j * SUB, SUB)], sem)
            for j in range(SPC)
        ]

        @plsc.parallel_loop(0, CHUNK // 16, unroll=2)
        def kfbody(i):
            x = dbuf[pl.ds(i * 16, 16)] * SCALE
            kv = x.astype(jnp.int32)
            f = x - kv.astype(jnp.float32)
            kbuf[pl.ds(i * 16, 16)] = kv
            fbuf[pl.ds(i * 16, 16)] = plsc.bitcast(
                plsc.pack(f, f, format=plsc.PackFormat.INTERLEAVED), jnp.int32)

        for cp in cps:
            cp.wait()

        @plsc.parallel_loop(0, CHUNK // 16, unroll=2)
        def mblock(m):
            eb = m * 16
            kv16 = kbuf[pl.ds(eb, 16)]
            fv16 = fbuf[pl.ds(eb, 16)]
            for i in range(16):
                e = eb + i
                krow = _lane_bcast(kv16, i)
                fp = plsc.bitcast(_lane_bcast(fv16, i), jnp.bfloat16)
                tp = plsc.load_gather(tbl_v, [krow, col0])
                dp = plsc.load_gather(tbl_v, [krow, col1])
                tv = plsc.bitcast(tp, jnp.bfloat16)
                dv = plsc.bitcast(dp, jnp.bfloat16)
                nv = plsc.bitcast(nbin[e, :], jnp.bfloat16)
                prod = nv * (tv + fp * dv)
                pa, pb = plsc.unpack(prod,
                                     format=plsc.PackFormat.INTERLEAVED,
                                     preferred_element_type=jnp.float32)
                nbuf[e, pl.ds(0, 16)] = pa
                nbuf[e, pl.ds(16, 16)] = pb
        for j in range(SPC):
            pltpu.sync_copy(nbuf.at[pl.ds(j * SUB, SUB)],
                            acc.at[idxd.at[j]], add=True)
        return carry

    lax.fori_loop(0, STEPS, ebody, 0)
    plsc.subcore_barrier()

    @pl.when(s < NS - 1)
    def _():
        pltpu.sync_copy(acc.at[pl.ds(r0, RPT)],
                        agg_hbm.at[c].at[pl.ds(r0, RPT)])

    @pl.when(s == NS - 1)
    def _():
        pltpu.sync_copy(acc.at[pl.ds(r0, RPT_LAST)],
                        agg_hbm.at[c].at[pl.ds(r0, RPT_LAST)])


@functools.lru_cache(maxsize=None)
def _edge_call(layer):
    mesh = plsc.VectorSubcoreMesh(core_axis_name="c", subcore_axis_name="s")
    return pl.kernel(
        functools.partial(_edge_body, layer),
        mesh=mesh,
        compiler_params=pltpu.CompilerParams(use_tc_tiling_on_sc=False,
                                             needs_layout_passes=False),
        out_type=jax.ShapeDtypeStruct((NC, N_NODES, HALF), jnp.float32),
        scratch_types=[
            pltpu.VMEM_SHARED((N_NODES, HALF), jnp.float32),
            pltpu.VMEM((KT, HALF), jnp.int32),
            pltpu.VMEM((SPC, SUB), jnp.int32),
            pltpu.VMEM((SPC, SUB), jnp.int32),
            pltpu.VMEM((CHUNK, HALF), jnp.float32),
            pltpu.VMEM((CHUNK, 16), jnp.int32),
            pltpu.VMEM((CHUNK,), jnp.float32),
            pltpu.VMEM((CHUNK,), jnp.int32),
            pltpu.VMEM((CHUNK,), jnp.int32),
            pltpu.SemaphoreType.DMA,
        ],
    )


def kernel(node_type, edge_index, distance, params):
    emb = params["embedding"]
    convs = params["convs"]
    table, embed, project, update, readout = _tc_calls(emb.shape[0])

    nt2 = node_type.astype(jnp.int32).reshape(N_NODES // _BN, 1, _BN)
    src = edge_index[0].astype(jnp.int32).reshape(N_EDGES // SUB, SUB)
    dst = edge_index[1].astype(jnp.int32).reshape(N_EDGES // SUB, SUB)

    w1s = jnp.stack([c["cf_w1"] for c in convs])
    b1s = jnp.stack([c["cf_b1"].reshape(DIM, 1) for c in convs])
    w2s = jnp.stack([c["cf_w2"] for c in convs])
    b2s = jnp.stack([c["cf_b2"] for c in convs])

    tbl = table(w1s, b1s, w2s, b2s)
    node = embed(nt2, emb)
    for l in range(N_CONV):
        nn = project(node, convs[l]["node_w1"])
        agg = _edge_call(l)(nn, tbl, distance, src, dst)
        node = update(node, agg, convs[l]["w2"], convs[l]["b2"],
                      convs[l]["w3"], convs[l]["b3"])
    return readout(node, params["d1_w"], params["d1_b"],
                   params["d2_w"], params["d2_b"])


# speedup vs baseline: 1.1034x; 1.1034x over previous
import functools

import jax
import jax.numpy as jnp
import numpy as np
from jax import lax
from jax.experimental import pallas as pl
from jax.experimental.pallas import tpu as pltpu
from jax.experimental.pallas import tpu_sc as plsc

N_NODES = 50000
N_EDGES = 800000
DIM = 64
N_CENTERS = 5
CUTOFF = 5.0
N_CONV = 3

NC = 2
NS = 16
HALF = 32

KT = 256
KINT = KT - 8
SCALE = KINT / CUTOFF

SUB = 100
SPC = 4
CHUNK = SUB * SPC
EPT = N_EDGES // NS
STEPS = EPT // CHUNK
ROWS_PT = EPT // SUB
RPT = 3128
RPT_LAST = N_NODES - (NS - 1) * RPT
ZFULL = RPT // CHUNK

_BCAST_DNUMS = lax.GatherDimensionNumbers(
    offset_dims=(), collapsed_slice_dims=(0,), start_index_map=(0,))


def _lane_bcast(v, i):
    idx = jnp.full((16, 1), i, jnp.int32)
    return lax.gather(v, idx, _BCAST_DNUMS, (1,),
                      mode=lax.GatherScatterMode.PROMISE_IN_BOUNDS)

_LOG2 = float(np.log(2.0))


def _bf16_bits(x):
    b = lax.bitcast_convert_type(x, jnp.int32)
    return lax.shift_right_logical(
        b + 0x7FFF + (lax.shift_right_logical(b, 16) & 1), 16)


def _pack_bf16(x, y):
    return _bf16_bits(x) | lax.shift_left(_bf16_bits(y), 16)


def _softplus(x):
    return jnp.maximum(x, 0.0) + jnp.log1p(jnp.exp(-jnp.abs(x)))


def _softplus_b(x):
    return 2.0 * _softplus(0.5 * x)


_BN = 5000

_TDOT = (((0,), (0,)), ((), ()))


def _table_body(w1_ref, b1_ref, w2_ref, b2_ref, out_ref):
    gap = CUTOFF / (N_CENTERS - 1)
    centers = lax.broadcasted_iota(
        jnp.int32, (N_CENTERS, 1), 0).astype(jnp.float32) * gap
    dk = lax.broadcasted_iota(
        jnp.int32, (1, KT), 1).astype(jnp.float32) * (1.0 / SCALE)
    rbfT = jnp.exp((-1.0 / gap) * (dk - centers) ** 2)
    for l in range(N_CONV):
        hpT = lax.dot_general(w1_ref[l], rbfT, _TDOT,
                              preferred_element_type=jnp.float32)
        hpT = _softplus_b(hpT + b1_ref[l])
        hh = lax.dot_general(hpT, w2_ref[l], _TDOT,
                             preferred_element_type=jnp.float32)
        hh = hh + b2_ref[l][None, :]
        dd = jnp.concatenate(
            [hh[1:] - hh[:-1], jnp.zeros((1, DIM), jnp.float32)], axis=0)
        for c in range(NC):
            t = hh[:, c * HALF:(c + 1) * HALF]
            d = dd[:, c * HALF:(c + 1) * HALF]
            dp = lax.bitcast_convert_type(
                _pack_bf16(d[:, :16], d[:, 16:]), jnp.float32)
            out_ref[l, c] = jnp.concatenate([t, dp], axis=1)


def _embed_body(nt_ref, emb_ref, out_ref):
    tn = emb_ref.shape[0]
    ids = lax.broadcasted_iota(jnp.int32, (tn, 1), 0)
    nt = nt_ref[0]
    ohT = (nt == ids).astype(jnp.float32)
    out_ref[...] = lax.dot_general(ohT, emb_ref[...], _TDOT,
                                   preferred_element_type=jnp.float32)


def _project_body(node_ref, w1_ref, out_ref):
    nn = jnp.dot(node_ref[...], w1_ref[...], preferred_element_type=jnp.float32)
    out_ref[0] = nn[:, :HALF]
    out_ref[1] = nn[:, HALF:]


def _update_body(node_ref, agg_ref, w2_ref, b2_ref, w3_ref, b3_ref, out_ref):
    agg = jnp.concatenate([agg_ref[0], agg_ref[1]], axis=1)
    cf = _softplus_b(
        jnp.dot(agg, w2_ref[...], preferred_element_type=jnp.float32)
        + b2_ref[...][None, :])
    out_ref[...] = (node_ref[...]
                    + jnp.dot(cf, w3_ref[...], preferred_element_type=jnp.float32)
                    + b3_ref[...][None, :])


def _readout_body(node_ref, d1w_ref, d1b_ref, d2w_ref, d2b_ref, out_ref):
    atom = _softplus(
        jnp.dot(node_ref[...], d1w_ref[...], preferred_element_type=jnp.float32)
        + d1b_ref[...][None, :]) - _LOG2
    res = jnp.dot(atom, d2w_ref[...], preferred_element_type=jnp.float32)
    part = jnp.sum(res) + node_ref.shape[0] * d2b_ref[0]

    @pl.when(pl.program_id(0) == 0)
    def _():
        out_ref[...] = jnp.zeros_like(out_ref)

    out_ref[...] += jnp.reshape(part, (1, 1))


def _full(shape):
    return pl.BlockSpec(shape, lambda i: tuple(0 for _ in shape))


@functools.lru_cache(maxsize=None)
def _tc_calls(type_num):
    table = pl.pallas_call(
        _table_body,
        grid=(1,),
        in_specs=[
            _full((N_CONV, N_CENTERS, DIM)),
            _full((N_CONV, DIM, 1)),
            _full((N_CONV, DIM, DIM)),
            _full((N_CONV, DIM)),
        ],
        out_specs=pl.BlockSpec((N_CONV, NC, KT, 48), lambda i: (0, 0, 0, 0)),
        out_shape=jax.ShapeDtypeStruct((N_CONV, NC, KT, 48), jnp.float32),
    )
    embed = pl.pallas_call(
        _embed_body,
        grid=(N_NODES // _BN,),
        in_specs=[
            pl.BlockSpec((1, 1, _BN), lambda i: (i, 0, 0)),
            _full((type_num, DIM)),
        ],
        out_specs=pl.BlockSpec((_BN, DIM), lambda i: (i, 0)),
        out_shape=jax.ShapeDtypeStruct((N_NODES, DIM), jnp.float32),
    )
    project = pl.pallas_call(
        _project_body,
        grid=(N_NODES // _BN,),
        in_specs=[
            pl.BlockSpec((_BN, DIM), lambda i: (i, 0)),
            _full((DIM, DIM)),
        ],
        out_specs=pl.BlockSpec((NC, _BN, HALF), lambda i: (0, i, 0)),
        out_shape=jax.ShapeDtypeStruct((NC, N_NODES, HALF), jnp.float32),
    )
    update = pl.pallas_call(
        _update_body,
        grid=(N_NODES // _BN,),
        in_specs=[
            pl.BlockSpec((_BN, DIM), lambda i: (i, 0)),
            pl.BlockSpec((NC, _BN, HALF), lambda i: (0, i, 0)),
            _full((DIM, DIM)),
            _full((DIM,)),
            _full((DIM, DIM)),
            _full((DIM,)),
        ],
        out_specs=pl.BlockSpec((_BN, DIM), lambda i: (i, 0)),
        out_shape=jax.ShapeDtypeStruct((N_NODES, DIM), jnp.float32),
    )
    readout = pl.pallas_call(
        _readout_body,
        grid=(N_NODES // _BN,),
        in_specs=[
            pl.BlockSpec((_BN, DIM), lambda i: (i, 0)),
            _full((DIM, DIM)),
            _full((DIM,)),
            _full((DIM, 1)),
            _full((1,)),
        ],
        out_specs=pl.BlockSpec((1, 1), lambda i: (0, 0)),
        out_shape=jax.ShapeDtypeStruct((1, 1), jnp.float32),
    )
    return table, embed, project, update, readout


def _edge_body(layer, nn_hbm, tbl_hbm, d_hbm, src_hbm, dst_hbm, agg_hbm,
               acc, tbl_v, idxs, idxd, nbuf, dbuf, kbuf, fbuf, sem):
    c = lax.axis_index("c")
    s = lax.axis_index("s")
    z16 = jnp.zeros((16,), jnp.float32)

    pltpu.sync_copy(tbl_hbm.at[layer].at[c], tbl_v)

    def zbody(i, carry):
        nbuf[i, pl.ds(0, 16)] = z16
        nbuf[i, pl.ds(16, 16)] = z16
        return carry

    lax.fori_loop(0, CHUNK, zbody, 0)
    r0 = s * RPT

    def zcopy(k, carry):
        pltpu.sync_copy(nbuf, acc.at[pl.ds(r0 + k * CHUNK, CHUNK)])
        return carry

    lax.fori_loop(0, ZFULL, zcopy, 0)

    @pl.when(s < NS - 1)
    def _():
        pltpu.sync_copy(nbuf.at[pl.ds(0, RPT - ZFULL * CHUNK)],
                        acc.at[pl.ds(r0 + ZFULL * CHUNK, RPT - ZFULL * CHUNK)])

    @pl.when(s == NS - 1)
    def _():
        pltpu.sync_copy(
            nbuf.at[pl.ds(0, RPT_LAST - ZFULL * CHUNK)],
            acc.at[pl.ds(r0 + ZFULL * CHUNK, RPT_LAST - ZFULL * CHUNK)])

    plsc.subcore_barrier()

    col0 = lax.broadcasted_iota(jnp.int32, (16,), 0)
    col1 = col0 + 16
    col2 = col0 + 32

    def ebody(k, carry):
        row0 = s * ROWS_PT + k * SPC
        e0 = s * EPT + k * CHUNK
        pltpu.sync_copy(src_hbm.at[pl.ds(row0, SPC)], idxs)
        pltpu.sync_copy(dst_hbm.at[pl.ds(row0, SPC)], idxd)
        pltpu.sync_copy(d_hbm.at[pl.ds(e0, CHUNK)], dbuf)
        cps = [
            pltpu.async_copy(nn_hbm.at[c].at[idxs.at[j]],
                             nbuf.at[pl.ds(j * SUB, SUB)], sem)
            for j in range(SPC)
        ]

        @plsc.parallel_loop(0, CHUNK // 16, unroll=2)
        def kfbody(i):
            x = dbuf[pl.ds(i * 16, 16)] * SCALE
            kv = x.astype(jnp.int32)
            kbuf[pl.ds(i * 16, 16)] = kv
            fbuf[pl.ds(i * 16, 16)] = x - kv.astype(jnp.float32)

        for cp in cps:
            cp.wait()

        @plsc.parallel_loop(0, CHUNK // 16, unroll=2)
        def mblock(m):
            eb = m * 16
            kv16 = kbuf[pl.ds(eb, 16)]
            fv16 = fbuf[pl.ds(eb, 16)]
            for i in range(16):
                e = eb + i
                krow = _lane_bcast(kv16, i)
                fv = _lane_bcast(fv16, i)
                ta = plsc.load_gather(tbl_v, [krow, col0])
                tb = plsc.load_gather(tbl_v, [krow, col1])
                dp = plsc.load_gather(tbl_v, [krow, col2])
                da, db = plsc.unpack(plsc.bitcast(dp, jnp.bfloat16),
                                     format=plsc.PackFormat.INTERLEAVED,
                                     preferred_element_type=jnp.float32)
                nbuf[e, pl.ds(0, 16)] = (
                    nbuf[e, pl.ds(0, 16)] * (ta + fv * da))
                nbuf[e, pl.ds(16, 16)] = (
                    nbuf[e, pl.ds(16, 16)] * (tb + fv * db))
        for j in range(SPC):
            pltpu.sync_copy(nbuf.at[pl.ds(j * SUB, SUB)],
                            acc.at[idxd.at[j]], add=True)
        return carry

    lax.fori_loop(0, STEPS, ebody, 0)
    plsc.subcore_barrier()

    @pl.when(s < NS - 1)
    def _():
        pltpu.sync_copy(acc.at[pl.ds(r0, RPT)],
                        agg_hbm.at[c].at[pl.ds(r0, RPT)])

    @pl.when(s == NS - 1)
    def _():
        pltpu.sync_copy(acc.at[pl.ds(r0, RPT_LAST)],
                        agg_hbm.at[c].at[pl.ds(r0, RPT_LAST)])


@functools.lru_cache(maxsize=None)
def _edge_call(layer):
    mesh = plsc.VectorSubcoreMesh(core_axis_name="c", subcore_axis_name="s")
    return pl.kernel(
        functools.partial(_edge_body, layer),
        mesh=mesh,
        compiler_params=pltpu.CompilerParams(use_tc_tiling_on_sc=False,
                                             needs_layout_passes=False),
        out_type=jax.ShapeDtypeStruct((NC, N_NODES, HALF), jnp.float32),
        scratch_types=[
            pltpu.VMEM_SHARED((N_NODES, HALF), jnp.float32),
            pltpu.VMEM((KT, 48), jnp.float32),
            pltpu.VMEM((SPC, SUB), jnp.int32),
            pltpu.VMEM((SPC, SUB), jnp.int32),
            pltpu.VMEM((CHUNK, HALF), jnp.float32),
            pltpu.VMEM((CHUNK,), jnp.float32),
            pltpu.VMEM((CHUNK,), jnp.int32),
            pltpu.VMEM((CHUNK,), jnp.float32),
            pltpu.SemaphoreType.DMA,
        ],
    )


def kernel(node_type, edge_index, distance, params):
    emb = params["embedding"]
    convs = params["convs"]
    table, embed, project, update, readout = _tc_calls(emb.shape[0])

    nt2 = node_type.astype(jnp.int32).reshape(N_NODES // _BN, 1, _BN)
    src = edge_index[0].astype(jnp.int32).reshape(N_EDGES // SUB, SUB)
    dst = edge_index[1].astype(jnp.int32).reshape(N_EDGES // SUB, SUB)

    w1s = jnp.stack([c["cf_w1"] for c in convs])
    b1s = jnp.stack([c["cf_b1"].reshape(DIM, 1) for c in convs])
    w2s = jnp.stack([c["cf_w2"] for c in convs])
    b2s = jnp.stack([c["cf_b2"] for c in convs])

    tbl = table(w1s, b1s, w2s, b2s)
    node = embed(nt2, emb)
    for l in range(N_CONV):
        nn = project(node, convs[l]["node_w1"])
        agg = _edge_call(l)(nn, tbl, distance, src, dst)
        node = update(node, agg, convs[l]["w2"], convs[l]["b2"],
                      convs[l]["w3"], convs[l]["b3"])
    return readout(node, params["d1_w"], params["d1_b"],
                   params["d2_w"], params["d2_b"])
